# baseline (device time: 131038 ns/iter reference)
import jax
import jax.numpy as jnp
from jax import lax
from jax.experimental import pallas as pl
from jax.experimental.pallas import tpu as pltpu

N_DEV = 4
M_BLK = 1024
K_BLK = 1024
N = 8192
NT = N // 1024
NSLOT = 2
NW = 36


def kernel(x, w_mat, scale_x, scale_w):
    def body(x_hbm, w_hbm, sx_ref, sw_ref, out_ref,
             stage_ref, xq_ref, xa0_ref, xa1_ref, wq_ref,
             stage_sems, send_sems, recv_sems):
        me = lax.axis_index("i")
        right = lax.rem(me + 1, N_DEV)
        opp = lax.rem(me + 2, N_DEV)
        left = lax.rem(me + 3, N_DEV)

        barrier_sem = pltpu.get_barrier_semaphore()
        for nbr in (right, opp, left):
            pl.semaphore_signal(barrier_sem, inc=1, device_id=(nbr,),
                                device_id_type=pl.DeviceIdType.MESH)
        pl.semaphore_wait(barrier_sem, 3)

        send_plan = [
            (right, xa0_ref.at[:, pl.ds(K_BLK, K_BLK)]),
            (left, xa1_ref.at[:, pl.ds(0, K_BLK)]),
            (opp, xa1_ref.at[:, pl.ds(K_BLK, K_BLK)]),
        ]
        x_order = [right, left, opp, me]
        k_pairs = [(me, left), (right, opp)]

        descs = []
        for i, blk in enumerate(x_order):
            descs.append(pltpu.make_async_copy(
                x_hbm.at[pl.ds(blk * M_BLK, M_BLK), :],
                stage_ref.at[i % NSLOT],
                stage_sems.at[i % NSLOT]))
        for j in range(32):
            p, t, m = j // 16, (j % 16) // 2, j % 2
            g = 4 + j
            descs.append(pltpu.make_async_copy(
                w_hbm.at[pl.ds(k_pairs[p][m] * K_BLK, K_BLK),
                         pl.ds(t * 1024, 1024)],
                stage_ref.at[g % NSLOT],
                stage_sems.at[g % NSLOT]))

        def pair_dot(xa, wb):
            return lax.dot_general(
                xa, wb, (((1,), (0,)), ((), ())),
                preferred_element_type=jnp.float32)

        def wait_recv(k):
            dst, dst_ref = send_plan[k]
            pltpu.make_async_remote_copy(
                src_ref=xq_ref.at[k],
                dst_ref=dst_ref,
                send_sem=send_sems.at[k],
                recv_sem=recv_sems.at[k],
                device_id=(me,),
                device_id_type=pl.DeviceIdType.MESH,
            ).wait_recv()

        sends = []
        for g in range(NSLOT):
            descs[g].start()
        for g in range(NW):
            descs[g].wait()
            if g < 4:
                if g < 3:
                    xq_ref[g] = stage_ref[g % NSLOT].astype(jnp.float8_e4m3fn)
                else:
                    xa0_ref[:, pl.ds(0, K_BLK)] = (
                        stage_ref[g % NSLOT].astype(jnp.float8_e4m3fn))
                if g + NSLOT < NW:
                    descs[g + NSLOT].start()
                if g < 3:
                    dst, dst_ref = send_plan[g]
                    rdma = pltpu.make_async_remote_copy(
                        src_ref=xq_ref.at[g],
                        dst_ref=dst_ref,
                        send_sem=send_sems.at[g],
                        recv_sem=recv_sems.at[g],
                        device_id=(dst,),
                        device_id_type=pl.DeviceIdType.MESH,
                    )
                    rdma.start()
                    sends.append(rdma)
            else:
                j = g - 4
                p, t, m = j // 16, (j % 16) // 2, j % 2
                wq_ref[t % 2, pl.ds(m * K_BLK, K_BLK), :] = (
                    stage_ref[g % NSLOT].astype(jnp.float8_e5m2))
                if g + NSLOT < NW:
                    descs[g + NSLOT].start()
                if m == 1:
                    cols = pl.ds(t * 1024, 1024)
                    if p == 0:
                        if t == 0:
                            wait_recv(1)
                        out_ref[:, cols] = pair_dot(xa0_ref[:, :],
                                                    wq_ref[t % 2])
                    else:
                        if t == 0:
                            wait_recv(0)
                            wait_recv(2)
                        y = (out_ref[:, cols]
                             + pair_dot(xa1_ref[:, :], wq_ref[t % 2]))
                        y = y * (sx_ref[0] * sw_ref[0])
                        out_ref[:, cols] = y * (1.0 / (1.0 + jnp.exp(-y)))

        for rdma in sends:
            rdma.wait_send()

    return pl.pallas_call(
        body,
        out_shape=jax.ShapeDtypeStruct((M_BLK, N), jnp.float32),
        in_specs=[
            pl.BlockSpec(memory_space=pl.ANY),
            pl.BlockSpec(memory_space=pl.ANY),
            pl.BlockSpec(memory_space=pltpu.SMEM),
            pl.BlockSpec(memory_space=pltpu.SMEM),
        ],
        out_specs=pl.BlockSpec(memory_space=pltpu.VMEM),
        scratch_shapes=[
            pltpu.VMEM((NSLOT, 1024, 1024), jnp.float32),
            pltpu.VMEM((3, M_BLK, K_BLK), jnp.float8_e4m3fn),
            pltpu.VMEM((M_BLK, 2 * K_BLK), jnp.float8_e4m3fn),
            pltpu.VMEM((M_BLK, 2 * K_BLK), jnp.float8_e4m3fn),
            pltpu.VMEM((2, 2 * K_BLK, 1024), jnp.float8_e5m2),
            pltpu.SemaphoreType.DMA((NSLOT,)),
            pltpu.SemaphoreType.DMA((3,)),
            pltpu.SemaphoreType.DMA((3,)),
        ],
        compiler_params=pltpu.CompilerParams(
            collective_id=0, vmem_limit_bytes=64 * 1024 * 1024),
    )(x, w_mat, scale_x, scale_w)
